# baseline (speedup 1.0000x reference)
import functools

import jax
import jax.numpy as jnp
from jax import lax
from jax.experimental import pallas as pl
from jax.experimental.pallas import tpu as pltpu
from jax.experimental.pallas import tpu_sc as plsc

N_NODES = 10000
N_EDGES = 320000
HID = 128

NC = 2
NS = 16
NW = NC * NS
CHUNK = 40
CPW = N_EDGES // NW // CHUNK
CHUNK_S = 40
CPW_S = N_EDGES // NW // CHUNK_S
NBUF_S = 2
NPAD = 10240
ROWS_PER_TILE = NPAD // NS


def _mesh():
    return plsc.VectorSubcoreMesh(core_axis_name="c", subcore_axis_name="s")


NBUF = 5


GCHUNK = 2 * CHUNK


def _sc_gather(ptab2, e2d):
    def body(ptab_h, e2d_h, gab_h, idx_v, buf, *sems):
        wid = lax.axis_index("s") * NC + lax.axis_index("c")
        pltpu.sync_copy(e2d_h.at[wid], idx_v)
        row0 = wid * CPW
        for b in range(NBUF):
            pltpu.async_copy(ptab_h.at[idx_v.at[b]], buf.at[b], sems[b])

        def group(g, carry):
            for b in range(NBUF):
                j = g * NBUF + b
                ebase = (row0 + j) * GCHUNK
                jn = j + NBUF
                pltpu.make_async_copy(
                    ptab_h.at[idx_v.at[j]], buf.at[b], sems[b]).wait()
                pltpu.sync_copy(buf.at[b], gab_h.at[pl.ds(ebase, GCHUNK)])

                @pl.when(jn < CPW)
                def _():
                    pltpu.async_copy(
                        ptab_h.at[idx_v.at[jn]], buf.at[b], sems[b])
            return carry

        lax.fori_loop(0, CPW // NBUF, group, 0)

    f = pl.kernel(
        body,
        out_type=jax.ShapeDtypeStruct((2 * N_EDGES, HID), jnp.float32),
        mesh=_mesh(),
        scratch_types=[
            pltpu.VMEM((CPW, GCHUNK), jnp.int32),
            pltpu.VMEM((NBUF, GCHUNK, HID), jnp.float32),
        ] + [pltpu.SemaphoreType.DMA] * NBUF,
        name="sc_gather",
    )
    return f(ptab2, e2d)


def _sc_scatter_add(edge, r2d, zeros):
    def body(edge_h, r2d_h, zeros_h, acc_h, ridx_v, ebuf, acc_s, *sems):
        cid = lax.axis_index("c")
        sid = lax.axis_index("s")
        wid = sid * NC + cid
        zbase = sid * ROWS_PER_TILE
        row0 = wid * CPW_S
        pltpu.sync_copy(r2d_h.at[wid], ridx_v)
        for b in range(NBUF_S):
            pltpu.async_copy(edge_h.at[pl.ds((row0 + b) * CHUNK_S, CHUNK_S)],
                             ebuf.at[b], sems[b])
        pltpu.sync_copy(zeros_h.at[pl.ds(zbase, ROWS_PER_TILE)],
                        acc_s.at[pl.ds(zbase, ROWS_PER_TILE)])
        plsc.subcore_barrier()

        def group(g, carry):
            for b in range(NBUF_S):
                j = g * NBUF_S + b
                jn = j + NBUF_S
                pltpu.make_async_copy(
                    edge_h.at[pl.ds((row0 + j) * CHUNK_S, CHUNK_S)],
                    ebuf.at[b], sems[b]).wait()
                pltpu.sync_copy(ebuf.at[b], acc_s.at[ridx_v.at[j]], add=True)

                @pl.when(jn < CPW_S)
                def _():
                    pltpu.async_copy(
                        edge_h.at[pl.ds((row0 + jn) * CHUNK_S, CHUNK_S)],
                        ebuf.at[b], sems[b])
            return carry

        lax.fori_loop(0, CPW_S // NBUF_S, group, 0)
        plsc.subcore_barrier()
        pltpu.sync_copy(acc_s.at[pl.ds(zbase, ROWS_PER_TILE)],
                        acc_h.at[cid].at[pl.ds(zbase, ROWS_PER_TILE)])

    f = pl.kernel(
        body,
        out_type=jax.ShapeDtypeStruct((NC, NPAD, HID), jnp.float32),
        mesh=_mesh(),
        scratch_types=[
            pltpu.VMEM((CPW_S, CHUNK_S), jnp.int32),
            pltpu.VMEM((NBUF_S, CHUNK_S, HID), jnp.float32),
            pltpu.VMEM_SHARED((NPAD, HID), jnp.float32),
        ] + [pltpu.SemaphoreType.DMA] * NBUF_S,
        name="sc_scatter_add",
    )
    return f(edge, r2d, zeros)


def _ln(y, g, be):
    m = jnp.mean(y, axis=-1, keepdims=True)
    v = jnp.mean((y - m) * (y - m), axis=-1, keepdims=True)
    return (y - m) * lax.rsqrt(v + 1e-5) * g + be


def _row_spec(blk, d):
    return pl.BlockSpec((blk, d), lambda i: (i, 0))


def _full_spec(a, b):
    return pl.BlockSpec((a, b), lambda i: (0, 0))


def _mlp_ln_tc(x, p, blk):
    rows, din = x.shape
    dh = p["W1"].shape[1]

    def body(x_ref, w1_ref, b1_ref, w2_ref, b2_ref, g_ref, be_ref, o_ref):
        h = jnp.dot(x_ref[...], w1_ref[...], preferred_element_type=jnp.float32)
        h = jnp.maximum(h + b1_ref[...], 0.0)
        y = jnp.dot(h, w2_ref[...], preferred_element_type=jnp.float32)
        o_ref[...] = _ln(y + b2_ref[...], g_ref[...], be_ref[...])

    return pl.pallas_call(
        body,
        grid=(rows // blk,),
        in_specs=[_row_spec(blk, din), _full_spec(din, dh), _full_spec(1, dh),
                  _full_spec(dh, dh), _full_spec(1, dh), _full_spec(1, dh),
                  _full_spec(1, dh)],
        out_specs=_row_spec(blk, dh),
        out_shape=jax.ShapeDtypeStruct((rows, dh), jnp.float32),
    )(x, p["W1"], p["b1"].reshape(1, dh), p["W2"], p["b2"].reshape(1, dh),
      p["g"].reshape(1, dh), p["be"].reshape(1, dh))


def _proj_tc(node, wcat, blk):
    rows = node.shape[0]

    def body(x_ref, w_ref, o1_ref, o2_ref, o3_ref):
        y = jnp.dot(x_ref[...], w_ref[...], preferred_element_type=jnp.float32)
        o1_ref[...] = y[:, :HID]
        o2_ref[...] = y[:, HID:2 * HID]
        o3_ref[...] = y[:, 2 * HID:]

    out = jax.ShapeDtypeStruct((rows, HID), jnp.float32)
    return pl.pallas_call(
        body,
        grid=(rows // blk,),
        in_specs=[_row_spec(blk, HID), _full_spec(HID, 3 * HID)],
        out_specs=(_row_spec(blk, HID),) * 3,
        out_shape=(out, out, out),
    )(node, wcat)


def _edge_step_tc(gab, edge, wc, p, blk):
    rows = edge.shape[0]

    def body(g_ref, e_ref, wc_ref, b1_ref, w2_ref, b2_ref, gg_ref, be_ref,
             o_ref):
        pre = jnp.dot(e_ref[...], wc_ref[...], preferred_element_type=jnp.float32)
        pre = pre + g_ref[:, 0] + g_ref[:, 1] + b1_ref[...]
        h = jnp.maximum(pre, 0.0)
        y = jnp.dot(h, w2_ref[...], preferred_element_type=jnp.float32)
        o_ref[...] = e_ref[...] + _ln(y + b2_ref[...], gg_ref[...], be_ref[...])

    return pl.pallas_call(
        body,
        grid=(rows // blk,),
        in_specs=[pl.BlockSpec((blk, 2, HID), lambda i: (i, 0, 0)),
                  _row_spec(blk, HID),
                  _full_spec(HID, HID), _full_spec(1, HID),
                  _full_spec(HID, HID), _full_spec(1, HID),
                  _full_spec(1, HID), _full_spec(1, HID)],
        out_specs=_row_spec(blk, HID),
        out_shape=jax.ShapeDtypeStruct((rows, HID), jnp.float32),
    )(gab.reshape(rows, 2, HID), edge, wc, p["b1"].reshape(1, HID), p["W2"],
      p["b2"].reshape(1, HID), p["g"].reshape(1, HID), p["be"].reshape(1, HID))


def _node_step_tc(acc2, node, nproj, wb, p, blk):
    rows = node.shape[0]

    def body(a_ref, n_ref, np_ref, wb_ref, b1_ref, w2_ref, b2_ref, g_ref,
             be_ref, o_ref):
        acc = a_ref[0] + a_ref[1]
        pre = jnp.dot(acc, wb_ref[...], preferred_element_type=jnp.float32)
        pre = pre + np_ref[...] + b1_ref[...]
        h = jnp.maximum(pre, 0.0)
        y = jnp.dot(h, w2_ref[...], preferred_element_type=jnp.float32)
        o_ref[...] = n_ref[...] + _ln(y + b2_ref[...], g_ref[...], be_ref[...])

    return pl.pallas_call(
        body,
        grid=(rows // blk,),
        in_specs=[pl.BlockSpec((NC, blk, HID), lambda i: (0, i, 0)),
                  _row_spec(blk, HID), _row_spec(blk, HID),
                  _full_spec(HID, HID), _full_spec(1, HID),
                  _full_spec(HID, HID), _full_spec(1, HID),
                  _full_spec(1, HID), _full_spec(1, HID)],
        out_specs=_row_spec(blk, HID),
        out_shape=jax.ShapeDtypeStruct((rows, HID), jnp.float32),
    )(acc2, node, nproj, wb, p["b1"].reshape(1, HID), p["W2"],
      p["b2"].reshape(1, HID), p["g"].reshape(1, HID), p["be"].reshape(1, HID))


def _dec_tc(node, p, blk):
    rows = node.shape[0]
    dout = p["W2"].shape[1]

    def body(x_ref, w1_ref, b1_ref, w2_ref, b2_ref, o_ref):
        h = jnp.dot(x_ref[...], w1_ref[...], preferred_element_type=jnp.float32)
        h = jnp.maximum(h + b1_ref[...], 0.0)
        y = jnp.dot(h, w2_ref[...], preferred_element_type=jnp.float32)
        o_ref[...] = y + b2_ref[...]

    return pl.pallas_call(
        body,
        grid=(rows // blk,),
        in_specs=[_row_spec(blk, HID), _full_spec(HID, HID), _full_spec(1, HID),
                  _full_spec(HID, dout), _full_spec(1, dout)],
        out_specs=_row_spec(blk, dout),
        out_shape=jax.ShapeDtypeStruct((rows, dout), jnp.float32),
    )(node, p["W1"], p["b1"].reshape(1, HID), p["W2"], p["b2"].reshape(1, dout))


N_STEPS = 3
NODE_BLK = 1000
EDGE_BLK = 2000


def kernel(edge_idx, node_feats, edge_feats, params):
    e2d = (edge_idx + jnp.array([[0, N_NODES]], jnp.int32)).reshape(
        NW, CPW, 2 * CHUNK)
    r2d_s = edge_idx[:, 1].reshape(NW, CPW_S, CHUNK_S)
    zeros = jnp.zeros((NPAD, HID), jnp.float32)

    def wcat(i):
        pe = params["mp%d_edge" % i]
        pn = params["mp%d_node" % i]
        return jnp.concatenate(
            [pe["W1"][:HID], pe["W1"][HID:2 * HID], pn["W1"][:HID]], axis=1)

    node = _mlp_ln_tc(node_feats, params["enc_node"], NODE_BLK)
    sproj, rproj, nproj = _proj_tc(node, wcat(0), NODE_BLK)
    edge = _mlp_ln_tc(edge_feats, params["enc_edge"], EDGE_BLK)

    for i in range(N_STEPS):
        pe = params["mp%d_edge" % i]
        pn = params["mp%d_node" % i]
        last = i == N_STEPS - 1
        if not last:
            gab = _sc_gather(jnp.concatenate([sproj, rproj], axis=0), e2d)
        acc2 = _sc_scatter_add(edge, r2d_s, zeros)
        if not last:
            edge = _edge_step_tc(gab, edge, pe["W1"][2 * HID:], pe, EDGE_BLK)
        node = _node_step_tc(acc2, node, nproj, pn["W1"][HID:], pn, NODE_BLK)
        if not last:
            sproj, rproj, nproj = _proj_tc(node, wcat(i + 1), NODE_BLK)

    return _dec_tc(node, params["dec"], NODE_BLK)


# speedup vs baseline: 1.1948x; 1.1948x over previous
import functools

import jax
import jax.numpy as jnp
from jax import lax
from jax.experimental import pallas as pl
from jax.experimental.pallas import tpu as pltpu
from jax.experimental.pallas import tpu_sc as plsc

N_NODES = 10000
N_EDGES = 320000
HID = 128

NC = 2
NS = 16
NW = NC * NS
CHUNK = 40
CPW = N_EDGES // NW // CHUNK
CHUNK_S = 40
CPW_S = N_EDGES // NW // CHUNK_S
NBUF_S = 2
NPAD = 10240
ROWS_PER_TILE = NPAD // NS


def _mesh():
    return plsc.VectorSubcoreMesh(core_axis_name="c", subcore_axis_name="s")


NBUF = 5


def _sc_gather(sproj, rproj, s2d, r2d):
    def body(sproj_h, rproj_h, s2d_h, r2d_h, ga_h, gb_h,
             sidx_v, ridx_v, buf_a, buf_b, *sems):
        sem_a, sem_b = sems[:NBUF], sems[NBUF:]
        wid = lax.axis_index("s") * NC + lax.axis_index("c")
        pltpu.sync_copy(s2d_h.at[wid], sidx_v)
        pltpu.sync_copy(r2d_h.at[wid], ridx_v)
        row0 = wid * CPW
        for b in range(NBUF):
            pltpu.async_copy(sproj_h.at[sidx_v.at[b]], buf_a.at[b], sem_a[b])
            pltpu.async_copy(rproj_h.at[ridx_v.at[b]], buf_b.at[b], sem_b[b])

        def group(g, carry):
            for b in range(NBUF):
                j = g * NBUF + b
                ebase = (row0 + j) * CHUNK
                jn = j + NBUF
                pltpu.make_async_copy(
                    sproj_h.at[sidx_v.at[j]], buf_a.at[b], sem_a[b]).wait()
                pltpu.sync_copy(buf_a.at[b], ga_h.at[pl.ds(ebase, CHUNK)])

                @pl.when(jn < CPW)
                def _():
                    pltpu.async_copy(
                        sproj_h.at[sidx_v.at[jn]], buf_a.at[b], sem_a[b])

                pltpu.make_async_copy(
                    rproj_h.at[ridx_v.at[j]], buf_b.at[b], sem_b[b]).wait()
                pltpu.sync_copy(buf_b.at[b], gb_h.at[pl.ds(ebase, CHUNK)])

                @pl.when(jn < CPW)
                def _():
                    pltpu.async_copy(
                        rproj_h.at[ridx_v.at[jn]], buf_b.at[b], sem_b[b])
            return carry

        lax.fori_loop(0, CPW // NBUF, group, 0)

    f = pl.kernel(
        body,
        out_type=(jax.ShapeDtypeStruct((N_EDGES, HID), jnp.float32),
                  jax.ShapeDtypeStruct((N_EDGES, HID), jnp.float32)),
        mesh=_mesh(),
        scratch_types=[
            pltpu.VMEM((CPW, CHUNK), jnp.int32),
            pltpu.VMEM((CPW, CHUNK), jnp.int32),
            pltpu.VMEM((NBUF, CHUNK, HID), jnp.float32),
            pltpu.VMEM((NBUF, CHUNK, HID), jnp.float32),
        ] + [pltpu.SemaphoreType.DMA] * (2 * NBUF),
        name="sc_gather",
    )
    return f(sproj, rproj, s2d, r2d)


def _sc_scatter_add(edge, r2d, zeros):
    def body(edge_h, r2d_h, zeros_h, acc_h, ridx_v, ebuf, acc_s, *sems):
        cid = lax.axis_index("c")
        sid = lax.axis_index("s")
        wid = sid * NC + cid
        zbase = sid * ROWS_PER_TILE
        row0 = wid * CPW_S
        pltpu.sync_copy(r2d_h.at[wid], ridx_v)
        for b in range(NBUF_S):
            pltpu.async_copy(edge_h.at[pl.ds((row0 + b) * CHUNK_S, CHUNK_S)],
                             ebuf.at[b], sems[b])
        pltpu.sync_copy(zeros_h.at[pl.ds(zbase, ROWS_PER_TILE)],
                        acc_s.at[pl.ds(zbase, ROWS_PER_TILE)])
        plsc.subcore_barrier()

        def group(g, carry):
            for b in range(NBUF_S):
                j = g * NBUF_S + b
                jn = j + NBUF_S
                pltpu.make_async_copy(
                    edge_h.at[pl.ds((row0 + j) * CHUNK_S, CHUNK_S)],
                    ebuf.at[b], sems[b]).wait()
                pltpu.sync_copy(ebuf.at[b], acc_s.at[ridx_v.at[j]], add=True)

                @pl.when(jn < CPW_S)
                def _():
                    pltpu.async_copy(
                        edge_h.at[pl.ds((row0 + jn) * CHUNK_S, CHUNK_S)],
                        ebuf.at[b], sems[b])
            return carry

        lax.fori_loop(0, CPW_S // NBUF_S, group, 0)
        plsc.subcore_barrier()
        pltpu.sync_copy(acc_s.at[pl.ds(zbase, ROWS_PER_TILE)],
                        acc_h.at[cid].at[pl.ds(zbase, ROWS_PER_TILE)])

    f = pl.kernel(
        body,
        out_type=jax.ShapeDtypeStruct((NC, NPAD, HID), jnp.float32),
        mesh=_mesh(),
        scratch_types=[
            pltpu.VMEM((CPW_S, CHUNK_S), jnp.int32),
            pltpu.VMEM((NBUF_S, CHUNK_S, HID), jnp.float32),
            pltpu.VMEM_SHARED((NPAD, HID), jnp.float32),
        ] + [pltpu.SemaphoreType.DMA] * NBUF_S,
        name="sc_scatter_add",
    )
    return f(edge, r2d, zeros)


def _ln(y, g, be):
    m = jnp.mean(y, axis=-1, keepdims=True)
    v = jnp.mean((y - m) * (y - m), axis=-1, keepdims=True)
    return (y - m) * lax.rsqrt(v + 1e-5) * g + be


def _row_spec(blk, d):
    return pl.BlockSpec((blk, d), lambda i: (i, 0))


def _full_spec(a, b):
    return pl.BlockSpec((a, b), lambda i: (0, 0))


def _mlp_ln_tc(x, p, blk):
    rows, din = x.shape
    dh = p["W1"].shape[1]

    def body(x_ref, w1_ref, b1_ref, w2_ref, b2_ref, g_ref, be_ref, o_ref):
        h = jnp.dot(x_ref[...], w1_ref[...], preferred_element_type=jnp.float32)
        h = jnp.maximum(h + b1_ref[...], 0.0)
        y = jnp.dot(h, w2_ref[...], preferred_element_type=jnp.float32)
        o_ref[...] = _ln(y + b2_ref[...], g_ref[...], be_ref[...])

    return pl.pallas_call(
        body,
        grid=(rows // blk,),
        in_specs=[_row_spec(blk, din), _full_spec(din, dh), _full_spec(1, dh),
                  _full_spec(dh, dh), _full_spec(1, dh), _full_spec(1, dh),
                  _full_spec(1, dh)],
        out_specs=_row_spec(blk, dh),
        out_shape=jax.ShapeDtypeStruct((rows, dh), jnp.float32),
    )(x, p["W1"], p["b1"].reshape(1, dh), p["W2"], p["b2"].reshape(1, dh),
      p["g"].reshape(1, dh), p["be"].reshape(1, dh))


def _proj_tc(node, wcat, blk):
    rows = node.shape[0]

    def body(x_ref, w_ref, o1_ref, o2_ref, o3_ref):
        y = jnp.dot(x_ref[...], w_ref[...], preferred_element_type=jnp.float32)
        o1_ref[...] = y[:, :HID]
        o2_ref[...] = y[:, HID:2 * HID]
        o3_ref[...] = y[:, 2 * HID:]

    out = jax.ShapeDtypeStruct((rows, HID), jnp.float32)
    return pl.pallas_call(
        body,
        grid=(rows // blk,),
        in_specs=[_row_spec(blk, HID), _full_spec(HID, 3 * HID)],
        out_specs=(_row_spec(blk, HID),) * 3,
        out_shape=(out, out, out),
    )(node, wcat)


def _edge_step_tc(ga, gb, edge, wc, p, blk):
    rows = edge.shape[0]

    def body(ga_ref, gb_ref, e_ref, wc_ref, b1_ref, w2_ref, b2_ref, g_ref,
             be_ref, o_ref):
        pre = jnp.dot(e_ref[...], wc_ref[...], preferred_element_type=jnp.float32)
        pre = pre + ga_ref[...] + gb_ref[...] + b1_ref[...]
        h = jnp.maximum(pre, 0.0)
        y = jnp.dot(h, w2_ref[...], preferred_element_type=jnp.float32)
        o_ref[...] = e_ref[...] + _ln(y + b2_ref[...], g_ref[...], be_ref[...])

    return pl.pallas_call(
        body,
        grid=(rows // blk,),
        in_specs=[_row_spec(blk, HID)] * 3 +
                 [_full_spec(HID, HID), _full_spec(1, HID),
                  _full_spec(HID, HID), _full_spec(1, HID),
                  _full_spec(1, HID), _full_spec(1, HID)],
        out_specs=_row_spec(blk, HID),
        out_shape=jax.ShapeDtypeStruct((rows, HID), jnp.float32),
    )(ga, gb, edge, wc, p["b1"].reshape(1, HID), p["W2"],
      p["b2"].reshape(1, HID), p["g"].reshape(1, HID), p["be"].reshape(1, HID))


def _node_step_tc(acc2, node, nproj, wb, p, blk):
    rows = node.shape[0]

    def body(a_ref, n_ref, np_ref, wb_ref, b1_ref, w2_ref, b2_ref, g_ref,
             be_ref, o_ref):
        acc = a_ref[0] + a_ref[1]
        pre = jnp.dot(acc, wb_ref[...], preferred_element_type=jnp.float32)
        pre = pre + np_ref[...] + b1_ref[...]
        h = jnp.maximum(pre, 0.0)
        y = jnp.dot(h, w2_ref[...], preferred_element_type=jnp.float32)
        o_ref[...] = n_ref[...] + _ln(y + b2_ref[...], g_ref[...], be_ref[...])

    return pl.pallas_call(
        body,
        grid=(rows // blk,),
        in_specs=[pl.BlockSpec((NC, blk, HID), lambda i: (0, i, 0)),
                  _row_spec(blk, HID), _row_spec(blk, HID),
                  _full_spec(HID, HID), _full_spec(1, HID),
                  _full_spec(HID, HID), _full_spec(1, HID),
                  _full_spec(1, HID), _full_spec(1, HID)],
        out_specs=_row_spec(blk, HID),
        out_shape=jax.ShapeDtypeStruct((rows, HID), jnp.float32),
    )(acc2, node, nproj, wb, p["b1"].reshape(1, HID), p["W2"],
      p["b2"].reshape(1, HID), p["g"].reshape(1, HID), p["be"].reshape(1, HID))


def _dec_tc(node, p, blk):
    rows = node.shape[0]
    dout = p["W2"].shape[1]

    def body(x_ref, w1_ref, b1_ref, w2_ref, b2_ref, o_ref):
        h = jnp.dot(x_ref[...], w1_ref[...], preferred_element_type=jnp.float32)
        h = jnp.maximum(h + b1_ref[...], 0.0)
        y = jnp.dot(h, w2_ref[...], preferred_element_type=jnp.float32)
        o_ref[...] = y + b2_ref[...]

    return pl.pallas_call(
        body,
        grid=(rows // blk,),
        in_specs=[_row_spec(blk, HID), _full_spec(HID, HID), _full_spec(1, HID),
                  _full_spec(HID, dout), _full_spec(1, dout)],
        out_specs=_row_spec(blk, dout),
        out_shape=jax.ShapeDtypeStruct((rows, dout), jnp.float32),
    )(node, p["W1"], p["b1"].reshape(1, HID), p["W2"], p["b2"].reshape(1, dout))


N_STEPS = 3
NODE_BLK = 2000
EDGE_BLK = 4000


def kernel(edge_idx, node_feats, edge_feats, params):
    s2d = edge_idx[:, 0].reshape(NW, CPW, CHUNK)
    r2d = edge_idx[:, 1].reshape(NW, CPW, CHUNK)
    r2d_s = edge_idx[:, 1].reshape(NW, CPW_S, CHUNK_S)
    zeros = jnp.zeros((NPAD, HID), jnp.float32)

    def wcat(i):
        pe = params["mp%d_edge" % i]
        pn = params["mp%d_node" % i]
        return jnp.concatenate(
            [pe["W1"][:HID], pe["W1"][HID:2 * HID], pn["W1"][:HID]], axis=1)

    node = _mlp_ln_tc(node_feats, params["enc_node"], NODE_BLK)
    sproj, rproj, nproj = _proj_tc(node, wcat(0), NODE_BLK)
    edge = _mlp_ln_tc(edge_feats, params["enc_edge"], EDGE_BLK)

    for i in range(N_STEPS):
        pe = params["mp%d_edge" % i]
        pn = params["mp%d_node" % i]
        last = i == N_STEPS - 1
        if not last:
            ga, gb = _sc_gather(sproj, rproj, s2d, r2d)
        acc2 = _sc_scatter_add(edge, r2d_s, zeros)
        if not last:
            edge = _edge_step_tc(ga, gb, edge, pe["W1"][2 * HID:], pe,
                                 EDGE_BLK)
        node = _node_step_tc(acc2, node, nproj, pn["W1"][HID:], pn, NODE_BLK)
        if not last:
            sproj, rproj, nproj = _proj_tc(node, wcat(i + 1), NODE_BLK)

    return _dec_tc(node, params["dec"], NODE_BLK)


# speedup vs baseline: 1.2129x; 1.0151x over previous
import functools

import jax
import jax.numpy as jnp
from jax import lax
from jax.experimental import pallas as pl
from jax.experimental.pallas import tpu as pltpu
from jax.experimental.pallas import tpu_sc as plsc

N_NODES = 10000
N_EDGES = 320000
HID = 128

NC = 2
NS = 16
NW = NC * NS
CHUNK = 40
CPW = N_EDGES // NW // CHUNK
CHUNK_S = 40
CPW_S = N_EDGES // NW // CHUNK_S
NBUF_S = 2
NPAD = 10240
ROWS_PER_TILE = NPAD // NS


def _mesh():
    return plsc.VectorSubcoreMesh(core_axis_name="c", subcore_axis_name="s")


NBUF = 5


def _sc_gather(sproj, rproj, s2d, r2d):
    def body(sproj_h, rproj_h, s2d_h, r2d_h, ga_h, gb_h,
             sidx_v, ridx_v, buf_a, buf_b, *sems):
        sem_a, sem_b = sems[:NBUF], sems[NBUF:]
        wid = lax.axis_index("s") * NC + lax.axis_index("c")
        pltpu.sync_copy(s2d_h.at[wid], sidx_v)
        pltpu.sync_copy(r2d_h.at[wid], ridx_v)
        row0 = wid * CPW
        for b in range(NBUF):
            pltpu.async_copy(sproj_h.at[sidx_v.at[b]], buf_a.at[b], sem_a[b])
            pltpu.async_copy(rproj_h.at[ridx_v.at[b]], buf_b.at[b], sem_b[b])

        def group(g, carry):
            for b in range(NBUF):
                j = g * NBUF + b
                ebase = (row0 + j) * CHUNK
                jn = j + NBUF
                pltpu.make_async_copy(
                    sproj_h.at[sidx_v.at[j]], buf_a.at[b], sem_a[b]).wait()
                pltpu.sync_copy(buf_a.at[b], ga_h.at[pl.ds(ebase, CHUNK)])

                @pl.when(jn < CPW)
                def _():
                    pltpu.async_copy(
                        sproj_h.at[sidx_v.at[jn]], buf_a.at[b], sem_a[b])

                pltpu.make_async_copy(
                    rproj_h.at[ridx_v.at[j]], buf_b.at[b], sem_b[b]).wait()
                pltpu.sync_copy(buf_b.at[b], gb_h.at[pl.ds(ebase, CHUNK)])

                @pl.when(jn < CPW)
                def _():
                    pltpu.async_copy(
                        rproj_h.at[ridx_v.at[jn]], buf_b.at[b], sem_b[b])
            return carry

        lax.fori_loop(0, CPW // NBUF, group, 0)

    f = pl.kernel(
        body,
        out_type=(jax.ShapeDtypeStruct((N_EDGES, HID), jnp.float32),
                  jax.ShapeDtypeStruct((N_EDGES, HID), jnp.float32)),
        mesh=_mesh(),
        scratch_types=[
            pltpu.VMEM((CPW, CHUNK), jnp.int32),
            pltpu.VMEM((CPW, CHUNK), jnp.int32),
            pltpu.VMEM((NBUF, CHUNK, HID), jnp.float32),
            pltpu.VMEM((NBUF, CHUNK, HID), jnp.float32),
        ] + [pltpu.SemaphoreType.DMA] * (2 * NBUF),
        name="sc_gather",
    )
    return f(sproj, rproj, s2d, r2d)


def _sc_scatter_add(edge, r2d, zeros):
    def body(edge_h, r2d_h, zeros_h, acc_h, ridx_v, ebuf, acc_s, *sems):
        cid = lax.axis_index("c")
        sid = lax.axis_index("s")
        wid = sid * NC + cid
        zbase = sid * ROWS_PER_TILE
        row0 = wid * CPW_S
        pltpu.sync_copy(r2d_h.at[wid], ridx_v)
        for b in range(NBUF_S):
            pltpu.async_copy(edge_h.at[pl.ds((row0 + b) * CHUNK_S, CHUNK_S)],
                             ebuf.at[b], sems[b])
        pltpu.sync_copy(zeros_h.at[pl.ds(zbase, ROWS_PER_TILE)],
                        acc_s.at[pl.ds(zbase, ROWS_PER_TILE)])
        plsc.subcore_barrier()

        def group(g, carry):
            for b in range(NBUF_S):
                j = g * NBUF_S + b
                jn = j + NBUF_S
                pltpu.make_async_copy(
                    edge_h.at[pl.ds((row0 + j) * CHUNK_S, CHUNK_S)],
                    ebuf.at[b], sems[b]).wait()
                pltpu.sync_copy(ebuf.at[b], acc_s.at[ridx_v.at[j]], add=True)

                @pl.when(jn < CPW_S)
                def _():
                    pltpu.async_copy(
                        edge_h.at[pl.ds((row0 + jn) * CHUNK_S, CHUNK_S)],
                        ebuf.at[b], sems[b])
            return carry

        lax.fori_loop(0, CPW_S // NBUF_S, group, 0)
        plsc.subcore_barrier()
        pltpu.sync_copy(acc_s.at[pl.ds(zbase, ROWS_PER_TILE)],
                        acc_h.at[cid].at[pl.ds(zbase, ROWS_PER_TILE)])

    f = pl.kernel(
        body,
        out_type=jax.ShapeDtypeStruct((NC, NPAD, HID), jnp.float32),
        mesh=_mesh(),
        scratch_types=[
            pltpu.VMEM((CPW_S, CHUNK_S), jnp.int32),
            pltpu.VMEM((NBUF_S, CHUNK_S, HID), jnp.float32),
            pltpu.VMEM_SHARED((NPAD, HID), jnp.float32),
        ] + [pltpu.SemaphoreType.DMA] * NBUF_S,
        name="sc_scatter_add",
    )
    return f(edge, r2d, zeros)


def _ln(y, g, be):
    m = jnp.mean(y, axis=-1, keepdims=True)
    v = jnp.mean((y - m) * (y - m), axis=-1, keepdims=True)
    return (y - m) * lax.rsqrt(v + 1e-5) * g + be


def _row_spec(blk, d):
    return pl.BlockSpec((blk, d), lambda i: (i, 0))


def _full_spec(a, b):
    return pl.BlockSpec((a, b), lambda i: (0, 0))


def _mlp_ln_tc(x, p, blk):
    rows, din = x.shape
    dh = p["W1"].shape[1]

    def body(x_ref, w1_ref, b1_ref, w2_ref, b2_ref, g_ref, be_ref, o_ref):
        h = jnp.dot(x_ref[...], w1_ref[...], preferred_element_type=jnp.float32)
        h = jnp.maximum(h + b1_ref[...], 0.0)
        y = jnp.dot(h, w2_ref[...], preferred_element_type=jnp.float32)
        o_ref[...] = _ln(y + b2_ref[...], g_ref[...], be_ref[...])

    return pl.pallas_call(
        body,
        grid=(rows // blk,),
        in_specs=[_row_spec(blk, din), _full_spec(din, dh), _full_spec(1, dh),
                  _full_spec(dh, dh), _full_spec(1, dh), _full_spec(1, dh),
                  _full_spec(1, dh)],
        out_specs=_row_spec(blk, dh),
        out_shape=jax.ShapeDtypeStruct((rows, dh), jnp.float32),
    )(x, p["W1"], p["b1"].reshape(1, dh), p["W2"], p["b2"].reshape(1, dh),
      p["g"].reshape(1, dh), p["be"].reshape(1, dh))


def _proj_tc(node, wcat, blk):
    rows = node.shape[0]

    def body(x_ref, w_ref, o1_ref, o2_ref, o3_ref):
        y = jnp.dot(x_ref[...], w_ref[...], preferred_element_type=jnp.float32)
        o1_ref[...] = y[:, :HID]
        o2_ref[...] = y[:, HID:2 * HID]
        o3_ref[...] = y[:, 2 * HID:]

    out = jax.ShapeDtypeStruct((rows, HID), jnp.float32)
    return pl.pallas_call(
        body,
        grid=(rows // blk,),
        in_specs=[_row_spec(blk, HID), _full_spec(HID, 3 * HID)],
        out_specs=(_row_spec(blk, HID),) * 3,
        out_shape=(out, out, out),
    )(node, wcat)


def _edge_step_tc(ga, gb, edge, wc, p, blk):
    rows = edge.shape[0]

    def body(ga_ref, gb_ref, e_ref, wc_ref, b1_ref, w2_ref, b2_ref, g_ref,
             be_ref, o_ref):
        pre = jnp.dot(e_ref[...], wc_ref[...], preferred_element_type=jnp.float32)
        pre = pre + ga_ref[...] + gb_ref[...] + b1_ref[...]
        h = jnp.maximum(pre, 0.0)
        y = jnp.dot(h, w2_ref[...], preferred_element_type=jnp.float32)
        o_ref[...] = e_ref[...] + _ln(y + b2_ref[...], g_ref[...], be_ref[...])

    return pl.pallas_call(
        body,
        grid=(rows // blk,),
        in_specs=[_row_spec(blk, HID)] * 3 +
                 [_full_spec(HID, HID), _full_spec(1, HID),
                  _full_spec(HID, HID), _full_spec(1, HID),
                  _full_spec(1, HID), _full_spec(1, HID)],
        out_specs=_row_spec(blk, HID),
        out_shape=jax.ShapeDtypeStruct((rows, HID), jnp.float32),
    )(ga, gb, edge, wc, p["b1"].reshape(1, HID), p["W2"],
      p["b2"].reshape(1, HID), p["g"].reshape(1, HID), p["be"].reshape(1, HID))


def _node_step_tc(acc2, node, nproj, wb, p, blk):
    rows = node.shape[0]

    def body(a_ref, n_ref, np_ref, wb_ref, b1_ref, w2_ref, b2_ref, g_ref,
             be_ref, o_ref):
        acc = a_ref[0] + a_ref[1]
        pre = jnp.dot(acc, wb_ref[...], preferred_element_type=jnp.float32)
        pre = pre + np_ref[...] + b1_ref[...]
        h = jnp.maximum(pre, 0.0)
        y = jnp.dot(h, w2_ref[...], preferred_element_type=jnp.float32)
        o_ref[...] = n_ref[...] + _ln(y + b2_ref[...], g_ref[...], be_ref[...])

    return pl.pallas_call(
        body,
        grid=(rows // blk,),
        in_specs=[pl.BlockSpec((NC, blk, HID), lambda i: (0, i, 0)),
                  _row_spec(blk, HID), _row_spec(blk, HID),
                  _full_spec(HID, HID), _full_spec(1, HID),
                  _full_spec(HID, HID), _full_spec(1, HID),
                  _full_spec(1, HID), _full_spec(1, HID)],
        out_specs=_row_spec(blk, HID),
        out_shape=jax.ShapeDtypeStruct((rows, HID), jnp.float32),
    )(acc2, node, nproj, wb, p["b1"].reshape(1, HID), p["W2"],
      p["b2"].reshape(1, HID), p["g"].reshape(1, HID), p["be"].reshape(1, HID))


def _dec_tc(node, p, blk):
    rows = node.shape[0]
    dout = p["W2"].shape[1]

    def body(x_ref, w1_ref, b1_ref, w2_ref, b2_ref, o_ref):
        h = jnp.dot(x_ref[...], w1_ref[...], preferred_element_type=jnp.float32)
        h = jnp.maximum(h + b1_ref[...], 0.0)
        y = jnp.dot(h, w2_ref[...], preferred_element_type=jnp.float32)
        o_ref[...] = y + b2_ref[...]

    return pl.pallas_call(
        body,
        grid=(rows // blk,),
        in_specs=[_row_spec(blk, HID), _full_spec(HID, HID), _full_spec(1, HID),
                  _full_spec(HID, dout), _full_spec(1, dout)],
        out_specs=_row_spec(blk, dout),
        out_shape=jax.ShapeDtypeStruct((rows, dout), jnp.float32),
    )(node, p["W1"], p["b1"].reshape(1, HID), p["W2"], p["b2"].reshape(1, dout))


N_STEPS = 3
NODE_BLK = 2000
EDGE_BLK = 8000


def kernel(edge_idx, node_feats, edge_feats, params):
    s2d = edge_idx[:, 0].reshape(NW, CPW, CHUNK)
    r2d = edge_idx[:, 1].reshape(NW, CPW, CHUNK)
    r2d_s = edge_idx[:, 1].reshape(NW, CPW_S, CHUNK_S)
    zeros = jnp.zeros((NPAD, HID), jnp.float32)

    def wcat(i):
        pe = params["mp%d_edge" % i]
        pn = params["mp%d_node" % i]
        return jnp.concatenate(
            [pe["W1"][:HID], pe["W1"][HID:2 * HID], pn["W1"][:HID]], axis=1)

    node = _mlp_ln_tc(node_feats, params["enc_node"], NODE_BLK)
    sproj, rproj, nproj = _proj_tc(node, wcat(0), NODE_BLK)
    edge = _mlp_ln_tc(edge_feats, params["enc_edge"], EDGE_BLK)

    for i in range(N_STEPS):
        pe = params["mp%d_edge" % i]
        pn = params["mp%d_node" % i]
        last = i == N_STEPS - 1
        if not last:
            ga, gb = _sc_gather(sproj, rproj, s2d, r2d)
        acc2 = _sc_scatter_add(edge, r2d_s, zeros)
        if not last:
            edge = _edge_step_tc(ga, gb, edge, pe["W1"][2 * HID:], pe,
                                 EDGE_BLK)
        node = _node_step_tc(acc2, node, nproj, pn["W1"][HID:], pn, NODE_BLK)
        if not last:
            sproj, rproj, nproj = _proj_tc(node, wcat(i + 1), NODE_BLK)

    return _dec_tc(node, params["dec"], NODE_BLK)


# speedup vs baseline: 1.2966x; 1.0691x over previous
import functools

import jax
import jax.numpy as jnp
from jax import lax
from jax.experimental import pallas as pl
from jax.experimental.pallas import tpu as pltpu
from jax.experimental.pallas import tpu_sc as plsc

N_NODES = 10000
N_EDGES = 320000
HID = 128

NC = 2
NS = 16
NW = NC * NS
CHUNK = 40
CPW = N_EDGES // NW // CHUNK
CHUNK_S = 80
CPW_S = N_EDGES // NW // CHUNK_S
NBUF_S = 2
NZTILES = 10
ZROWS = N_NODES // NZTILES


def _mesh():
    return plsc.VectorSubcoreMesh(core_axis_name="c", subcore_axis_name="s")


NBUF = 5


def _sc_gather(sproj, rproj, s2d, r2d):
    def body(sproj_h, rproj_h, s2d_h, r2d_h, ga_h, gb_h,
             sidx_v, ridx_v, buf_a, buf_b, *sems):
        sem_a, sem_b = sems[:NBUF], sems[NBUF:]
        wid = lax.axis_index("s") * NC + lax.axis_index("c")
        pltpu.sync_copy(s2d_h.at[wid], sidx_v)
        pltpu.sync_copy(r2d_h.at[wid], ridx_v)
        row0 = wid * CPW
        for b in range(NBUF):
            pltpu.async_copy(sproj_h.at[sidx_v.at[b]], buf_a.at[b], sem_a[b])
            pltpu.async_copy(rproj_h.at[ridx_v.at[b]], buf_b.at[b], sem_b[b])

        def group(g, carry):
            for b in range(NBUF):
                j = g * NBUF + b
                ebase = (row0 + j) * CHUNK
                jn = j + NBUF
                pltpu.make_async_copy(
                    sproj_h.at[sidx_v.at[j]], buf_a.at[b], sem_a[b]).wait()
                pltpu.sync_copy(buf_a.at[b], ga_h.at[pl.ds(ebase, CHUNK)])

                @pl.when(jn < CPW)
                def _():
                    pltpu.async_copy(
                        sproj_h.at[sidx_v.at[jn]], buf_a.at[b], sem_a[b])

                pltpu.make_async_copy(
                    rproj_h.at[ridx_v.at[j]], buf_b.at[b], sem_b[b]).wait()
                pltpu.sync_copy(buf_b.at[b], gb_h.at[pl.ds(ebase, CHUNK)])

                @pl.when(jn < CPW)
                def _():
                    pltpu.async_copy(
                        rproj_h.at[ridx_v.at[jn]], buf_b.at[b], sem_b[b])
            return carry

        lax.fori_loop(0, CPW // NBUF, group, 0)

    f = pl.kernel(
        body,
        out_type=(jax.ShapeDtypeStruct((N_EDGES, HID), jnp.float32),
                  jax.ShapeDtypeStruct((N_EDGES, HID), jnp.float32)),
        mesh=_mesh(),
        scratch_types=[
            pltpu.VMEM((CPW, CHUNK), jnp.int32),
            pltpu.VMEM((CPW, CHUNK), jnp.int32),
            pltpu.VMEM((NBUF, CHUNK, HID), jnp.float32),
            pltpu.VMEM((NBUF, CHUNK, HID), jnp.float32),
        ] + [pltpu.SemaphoreType.DMA] * (2 * NBUF),
        name="sc_gather",
    )
    return f(sproj, rproj, s2d, r2d)


def _sc_scatter_add(edge, r2d, zeros):
    ngroups = -(-CPW_S // NBUF_S)

    def body(edge_h, r2d_h, zeros_h, acc_h, ibuf, ebuf, acc_s, *sems):
        isem = sems[:NBUF_S]
        esem = sems[NBUF_S:]
        cid = lax.axis_index("c")
        sid = lax.axis_index("s")
        wid = sid * NC + cid
        row0 = wid * CPW_S
        for b in range(NBUF_S):
            pltpu.async_copy(r2d_h.at[row0 + b], ibuf.at[b], isem[b])
            pltpu.async_copy(edge_h.at[pl.ds((row0 + b) * CHUNK_S, CHUNK_S)],
                             ebuf.at[b], esem[b])

        @pl.when(sid < NZTILES)
        def _():
            zbase = sid * ZROWS
            pltpu.sync_copy(zeros_h.at[pl.ds(zbase, ZROWS)],
                            acc_s.at[pl.ds(zbase, ZROWS)])

        plsc.subcore_barrier()

        def group(g, carry):
            for b in range(NBUF_S):
                j = g * NBUF_S + b
                jn = j + NBUF_S

                @pl.when(j < CPW_S)
                def _():
                    pltpu.make_async_copy(
                        r2d_h.at[row0 + j], ibuf.at[b], isem[b]).wait()
                    pltpu.make_async_copy(
                        edge_h.at[pl.ds((row0 + j) * CHUNK_S, CHUNK_S)],
                        ebuf.at[b], esem[b]).wait()
                    pltpu.sync_copy(ebuf.at[b], acc_s.at[ibuf.at[b]],
                                    add=True)

                @pl.when(jn < CPW_S)
                def _():
                    pltpu.async_copy(r2d_h.at[row0 + jn], ibuf.at[b], isem[b])
                    pltpu.async_copy(
                        edge_h.at[pl.ds((row0 + jn) * CHUNK_S, CHUNK_S)],
                        ebuf.at[b], esem[b])
            return carry

        lax.fori_loop(0, ngroups, group, 0)
        plsc.subcore_barrier()

        @pl.when(sid < NZTILES)
        def _():
            zbase = sid * ZROWS
            pltpu.sync_copy(acc_s.at[pl.ds(zbase, ZROWS)],
                            acc_h.at[cid].at[pl.ds(zbase, ZROWS)])

    f = pl.kernel(
        body,
        out_type=jax.ShapeDtypeStruct((NC, N_NODES, HID), jnp.float32),
        mesh=_mesh(),
        scratch_types=[
            pltpu.VMEM((NBUF_S, CHUNK_S), jnp.int32),
            pltpu.VMEM((NBUF_S, CHUNK_S, HID), jnp.float32),
            pltpu.VMEM_SHARED((N_NODES, HID), jnp.float32),
        ] + [pltpu.SemaphoreType.DMA] * (2 * NBUF_S),
        name="sc_scatter_add",
    )
    return f(edge, r2d, zeros)


def _ln(y, g, be):
    m = jnp.mean(y, axis=-1, keepdims=True)
    v = jnp.mean((y - m) * (y - m), axis=-1, keepdims=True)
    return (y - m) * lax.rsqrt(v + 1e-5) * g + be


def _row_spec(blk, d):
    return pl.BlockSpec((blk, d), lambda i: (i, 0))


def _full_spec(a, b):
    return pl.BlockSpec((a, b), lambda i: (0, 0))


def _mlp_ln_tc(x, p, blk):
    rows, din = x.shape
    dh = p["W1"].shape[1]

    def body(x_ref, w1_ref, b1_ref, w2_ref, b2_ref, g_ref, be_ref, o_ref):
        h = jnp.dot(x_ref[...], w1_ref[...], preferred_element_type=jnp.float32)
        h = jnp.maximum(h + b1_ref[...], 0.0)
        y = jnp.dot(h, w2_ref[...], preferred_element_type=jnp.float32)
        o_ref[...] = _ln(y + b2_ref[...], g_ref[...], be_ref[...])

    return pl.pallas_call(
        body,
        grid=(rows // blk,),
        in_specs=[_row_spec(blk, din), _full_spec(din, dh), _full_spec(1, dh),
                  _full_spec(dh, dh), _full_spec(1, dh), _full_spec(1, dh),
                  _full_spec(1, dh)],
        out_specs=_row_spec(blk, dh),
        out_shape=jax.ShapeDtypeStruct((rows, dh), jnp.float32),
    )(x, p["W1"], p["b1"].reshape(1, dh), p["W2"], p["b2"].reshape(1, dh),
      p["g"].reshape(1, dh), p["be"].reshape(1, dh))


def _proj_tc(node, wcat, blk):
    rows = node.shape[0]

    def body(x_ref, w_ref, o1_ref, o2_ref, o3_ref):
        y = jnp.dot(x_ref[...], w_ref[...], preferred_element_type=jnp.float32)
        o1_ref[...] = y[:, :HID]
        o2_ref[...] = y[:, HID:2 * HID]
        o3_ref[...] = y[:, 2 * HID:]

    out = jax.ShapeDtypeStruct((rows, HID), jnp.float32)
    return pl.pallas_call(
        body,
        grid=(rows // blk,),
        in_specs=[_row_spec(blk, HID), _full_spec(HID, 3 * HID)],
        out_specs=(_row_spec(blk, HID),) * 3,
        out_shape=(out, out, out),
    )(node, wcat)


def _edge_step_tc(ga, gb, edge, wc, p, blk):
    rows = edge.shape[0]

    def body(ga_ref, gb_ref, e_ref, wc_ref, b1_ref, w2_ref, b2_ref, g_ref,
             be_ref, o_ref):
        pre = jnp.dot(e_ref[...], wc_ref[...], preferred_element_type=jnp.float32)
        pre = pre + ga_ref[...] + gb_ref[...] + b1_ref[...]
        h = jnp.maximum(pre, 0.0)
        y = jnp.dot(h, w2_ref[...], preferred_element_type=jnp.float32)
        o_ref[...] = e_ref[...] + _ln(y + b2_ref[...], g_ref[...], be_ref[...])

    return pl.pallas_call(
        body,
        grid=(rows // blk,),
        in_specs=[_row_spec(blk, HID)] * 3 +
                 [_full_spec(HID, HID), _full_spec(1, HID),
                  _full_spec(HID, HID), _full_spec(1, HID),
                  _full_spec(1, HID), _full_spec(1, HID)],
        out_specs=_row_spec(blk, HID),
        out_shape=jax.ShapeDtypeStruct((rows, HID), jnp.float32),
    )(ga, gb, edge, wc, p["b1"].reshape(1, HID), p["W2"],
      p["b2"].reshape(1, HID), p["g"].reshape(1, HID), p["be"].reshape(1, HID))


def _node_step_tc(acc2, node, nproj, wb, p, blk):
    rows = node.shape[0]

    def body(a_ref, n_ref, np_ref, wb_ref, b1_ref, w2_ref, b2_ref, g_ref,
             be_ref, o_ref):
        acc = a_ref[0] + a_ref[1]
        pre = jnp.dot(acc, wb_ref[...], preferred_element_type=jnp.float32)
        pre = pre + np_ref[...] + b1_ref[...]
        h = jnp.maximum(pre, 0.0)
        y = jnp.dot(h, w2_ref[...], preferred_element_type=jnp.float32)
        o_ref[...] = n_ref[...] + _ln(y + b2_ref[...], g_ref[...], be_ref[...])

    return pl.pallas_call(
        body,
        grid=(rows // blk,),
        in_specs=[pl.BlockSpec((NC, blk, HID), lambda i: (0, i, 0)),
                  _row_spec(blk, HID), _row_spec(blk, HID),
                  _full_spec(HID, HID), _full_spec(1, HID),
                  _full_spec(HID, HID), _full_spec(1, HID),
                  _full_spec(1, HID), _full_spec(1, HID)],
        out_specs=_row_spec(blk, HID),
        out_shape=jax.ShapeDtypeStruct((rows, HID), jnp.float32),
    )(acc2, node, nproj, wb, p["b1"].reshape(1, HID), p["W2"],
      p["b2"].reshape(1, HID), p["g"].reshape(1, HID), p["be"].reshape(1, HID))


def _dec_tc(node, p, blk):
    rows = node.shape[0]
    dout = p["W2"].shape[1]

    def body(x_ref, w1_ref, b1_ref, w2_ref, b2_ref, o_ref):
        h = jnp.dot(x_ref[...], w1_ref[...], preferred_element_type=jnp.float32)
        h = jnp.maximum(h + b1_ref[...], 0.0)
        y = jnp.dot(h, w2_ref[...], preferred_element_type=jnp.float32)
        o_ref[...] = y + b2_ref[...]

    return pl.pallas_call(
        body,
        grid=(rows // blk,),
        in_specs=[_row_spec(blk, HID), _full_spec(HID, HID), _full_spec(1, HID),
                  _full_spec(HID, dout), _full_spec(1, dout)],
        out_specs=_row_spec(blk, dout),
        out_shape=jax.ShapeDtypeStruct((rows, dout), jnp.float32),
    )(node, p["W1"], p["b1"].reshape(1, HID), p["W2"], p["b2"].reshape(1, dout))


N_STEPS = 3
NODE_BLK = 2000
EDGE_BLK = 8000


def kernel(edge_idx, node_feats, edge_feats, params):
    s2d = edge_idx[:, 0].reshape(NW, CPW, CHUNK)
    r2d = edge_idx[:, 1].reshape(NW, CPW, CHUNK)
    r2d_s = edge_idx[:, 1].reshape(NW * CPW_S, CHUNK_S)
    zeros = jnp.zeros((N_NODES, HID), jnp.float32)

    def wcat(i):
        pe = params["mp%d_edge" % i]
        pn = params["mp%d_node" % i]
        return jnp.concatenate(
            [pe["W1"][:HID], pe["W1"][HID:2 * HID], pn["W1"][:HID]], axis=1)

    node = _mlp_ln_tc(node_feats, params["enc_node"], NODE_BLK)
    sproj, rproj, nproj = _proj_tc(node, wcat(0), NODE_BLK)
    edge = _mlp_ln_tc(edge_feats, params["enc_edge"], EDGE_BLK)

    for i in range(N_STEPS):
        pe = params["mp%d_edge" % i]
        pn = params["mp%d_node" % i]
        last = i == N_STEPS - 1
        if not last:
            ga, gb = _sc_gather(sproj, rproj, s2d, r2d)
        acc2 = _sc_scatter_add(edge, r2d_s, zeros)
        if not last:
            edge = _edge_step_tc(ga, gb, edge, pe["W1"][2 * HID:], pe,
                                 EDGE_BLK)
        node = _node_step_tc(acc2, node, nproj, pn["W1"][HID:], pn, NODE_BLK)
        if not last:
            sproj, rproj, nproj = _proj_tc(node, wcat(i + 1), NODE_BLK)

    return _dec_tc(node, params["dec"], NODE_BLK)
